# baseline (speedup 1.0000x reference)
import jax
import jax.numpy as jnp
from jax.experimental import pallas as pl

_D = 768
_E = 8
_R = 8
_ER = _E * _R
_CC = 128
_SCALING = 16.0 / 8.0
_MAX_THRESHOLD = 0.125
_TB = 2048


def _fused_body(x_ref, wb_ref, bb_ref, wcat_ref, bt_ref, bc_ref, out_ref):
    xb = x_ref[...]
    xh = xb.astype(jnp.bfloat16)
    base = jnp.dot(xh, wb_ref[...], preferred_element_type=jnp.float32)
    base = base + bb_ref[...]

    cat = jnp.dot(xh, wcat_ref[...], preferred_element_type=jnp.float32)
    h = cat[:, :_ER]
    gl = cat[:, _ER:_ER + _E]
    tl = cat[:, _ER + _E:_ER + _E + 1]

    e = jnp.exp(gl)
    s = jnp.sum(e, axis=-1, keepdims=True)
    thr = jax.nn.sigmoid(tl + bt_ref[0, 0]) * _MAX_THRESHOLD
    u = e - thr * s
    u = jnp.where(u >= 0.0, u, 0.0)
    denom = jnp.sum(u, axis=-1, keepdims=True)
    denom = jnp.where(denom == 0.0, 1.0, denom)
    w = u / denom

    rows = jax.lax.broadcasted_iota(jnp.int32, (_E, _ER), 0)
    cols = jax.lax.broadcasted_iota(jnp.int32, (_E, _ER), 1)
    expand = (cols // _R == rows).astype(jnp.float32)
    wrep = jnp.dot(w, expand, preferred_element_type=jnp.float32)

    hw = (h * wrep).astype(jnp.bfloat16)
    moe = jnp.dot(hw, bc_ref[...], preferred_element_type=jnp.float32)
    out_ref[...] = base + moe


def kernel(x, W_base, b_base, W_gate, W_thr, b_thr, A, Bm):
    d = x.shape[-1]
    flat = x.reshape(-1, d)
    t = flat.shape[0]
    a_cat = A.transpose(1, 0, 2).reshape(d, _ER)
    pad = jnp.zeros((d, _CC - _ER - _E - 1), dtype=x.dtype)
    w_cat = jnp.concatenate([a_cat, W_gate, W_thr, pad],
                            axis=1).astype(jnp.bfloat16)
    b_cat = (Bm.reshape(_ER, d) * _SCALING).astype(jnp.bfloat16)
    wb_h = W_base.astype(jnp.bfloat16)
    out = pl.pallas_call(
        _fused_body,
        grid=(t // _TB,),
        in_specs=[
            pl.BlockSpec((_TB, d), lambda i: (i, 0)),
            pl.BlockSpec((d, d), lambda i: (0, 0)),
            pl.BlockSpec((1, d), lambda i: (0, 0)),
            pl.BlockSpec((d, _CC), lambda i: (0, 0)),
            pl.BlockSpec((1, 1), lambda i: (0, 0)),
            pl.BlockSpec((_ER, d), lambda i: (0, 0)),
        ],
        out_specs=pl.BlockSpec((_TB, d), lambda i: (i, 0)),
        out_shape=jax.ShapeDtypeStruct((t, d), jnp.float32),
    )(flat, wb_h, b_base.reshape(1, d), w_cat, b_thr.reshape(1, 1), b_cat)
    return out.reshape(x.shape)


# speedup vs baseline: 1.3094x; 1.3094x over previous
import jax
import jax.numpy as jnp
from jax.experimental import pallas as pl

_D = 768
_E = 8
_R = 8
_ER = _E * _R
_SCALING = 16.0 / 8.0
_MAX_THRESHOLD = 0.125
_TB = 2048


def _fused_body(x_ref, wb_ref, bb_ref, wg_ref, wt_ref, bt_ref, ac_ref,
                bc_ref, out_ref):
    xb = x_ref[...]
    xh = xb.astype(jnp.bfloat16)
    base = jnp.dot(xh, wb_ref[...].astype(jnp.bfloat16),
                   preferred_element_type=jnp.float32)
    base = base + bb_ref[...]

    gl = jnp.dot(xh, wg_ref[...].astype(jnp.bfloat16),
                 preferred_element_type=jnp.float32)
    tl = jnp.dot(xh, wt_ref[...].astype(jnp.bfloat16),
                 preferred_element_type=jnp.float32)
    e = jnp.exp(gl)
    s = jnp.sum(e, axis=-1, keepdims=True)
    thr = jax.nn.sigmoid(tl + bt_ref[0, 0]) * _MAX_THRESHOLD
    u = e - thr * s
    u = jnp.where(u >= 0.0, u, 0.0)
    denom = jnp.sum(u, axis=-1, keepdims=True)
    denom = jnp.where(denom == 0.0, 1.0, denom)
    w = u / denom

    rows = jax.lax.broadcasted_iota(jnp.int32, (_E, _ER), 0)
    cols = jax.lax.broadcasted_iota(jnp.int32, (_E, _ER), 1)
    expand = (cols // _R == rows).astype(jnp.float32)
    wrep = jnp.dot(w, expand, preferred_element_type=jnp.float32)

    h = jnp.dot(xh, ac_ref[...].astype(jnp.bfloat16),
                preferred_element_type=jnp.float32)
    hw = (h * (wrep * _SCALING)).astype(jnp.bfloat16)
    moe = jnp.dot(hw, bc_ref[...].astype(jnp.bfloat16),
                  preferred_element_type=jnp.float32)
    out_ref[...] = base + moe


def kernel(x, W_base, b_base, W_gate, W_thr, b_thr, A, Bm):
    d = x.shape[-1]
    flat = x.reshape(-1, d)
    t = flat.shape[0]
    a_cat = A.transpose(1, 0, 2).reshape(d, _ER)
    b_cat = Bm.reshape(_ER, d)
    wb_h = W_base
    wg_h = W_gate
    wt_h = W_thr
    out = pl.pallas_call(
        _fused_body,
        grid=(t // _TB,),
        in_specs=[
            pl.BlockSpec((_TB, d), lambda i: (i, 0)),
            pl.BlockSpec((d, d), lambda i: (0, 0)),
            pl.BlockSpec((1, d), lambda i: (0, 0)),
            pl.BlockSpec((d, _E), lambda i: (0, 0)),
            pl.BlockSpec((d, 1), lambda i: (0, 0)),
            pl.BlockSpec((1, 1), lambda i: (0, 0)),
            pl.BlockSpec((d, _ER), lambda i: (0, 0)),
            pl.BlockSpec((_ER, d), lambda i: (0, 0)),
        ],
        out_specs=pl.BlockSpec((_TB, d), lambda i: (i, 0)),
        out_shape=jax.ShapeDtypeStruct((t, d), jnp.float32),
    )(flat, wb_h, b_base.reshape(1, d), wg_h, wt_h,
      b_thr.reshape(1, 1), a_cat, b_cat)
    return out.reshape(x.shape)


# speedup vs baseline: 1.5648x; 1.1950x over previous
import jax
import jax.numpy as jnp
from jax.experimental import pallas as pl

_D = 768
_E = 8
_R = 8
_ER = _E * _R
_SCALING = 16.0 / 8.0
_MAX_THRESHOLD = 0.125
_TB = 2048


def _fused_body(x_ref, wb_ref, bb_ref, wgt_ref, bt_ref, ac_ref,
                bc_ref, out_ref):
    xb = x_ref[...]
    xh = xb.astype(jnp.bfloat16)
    base = jnp.dot(xh, wb_ref[...].astype(jnp.bfloat16),
                   preferred_element_type=jnp.float32)
    base = base + bb_ref[...]

    gt = jnp.dot(xh, wgt_ref[...].astype(jnp.bfloat16),
                 preferred_element_type=jnp.float32)
    gl = gt[:, :_E]
    tl = gt[:, _E:_E + 1]
    e = jnp.exp(gl)
    s = jnp.sum(e, axis=-1, keepdims=True)
    thr = jax.nn.sigmoid(tl + bt_ref[0, 0]) * _MAX_THRESHOLD
    u = e - thr * s
    u = jnp.where(u >= 0.0, u, 0.0)
    denom = jnp.sum(u, axis=-1, keepdims=True)
    denom = jnp.where(denom == 0.0, 1.0, denom)
    w = u / denom

    rows = jax.lax.broadcasted_iota(jnp.int32, (_E, _ER), 0)
    cols = jax.lax.broadcasted_iota(jnp.int32, (_E, _ER), 1)
    expand = (cols // _R == rows).astype(jnp.float32)
    wrep = jnp.dot(w, expand, preferred_element_type=jnp.float32)

    h = jnp.dot(xh, ac_ref[...].astype(jnp.bfloat16),
                preferred_element_type=jnp.float32)
    hw = (h * (wrep * _SCALING)).astype(jnp.bfloat16)
    moe = jnp.dot(hw, bc_ref[...].astype(jnp.bfloat16),
                  preferred_element_type=jnp.float32)
    out_ref[...] = base + moe


def kernel(x, W_base, b_base, W_gate, W_thr, b_thr, A, Bm):
    d = x.shape[-1]
    flat = x.reshape(-1, d)
    t = flat.shape[0]
    a_cat = A.transpose(1, 0, 2).reshape(d, _ER)
    b_cat = Bm.reshape(_ER, d)
    w_gt = jnp.concatenate([W_gate, W_thr], axis=1)
    out = pl.pallas_call(
        _fused_body,
        grid=(t // _TB,),
        in_specs=[
            pl.BlockSpec((_TB, d), lambda i: (i, 0)),
            pl.BlockSpec((d, d), lambda i: (0, 0)),
            pl.BlockSpec((1, d), lambda i: (0, 0)),
            pl.BlockSpec((d, _E + 1), lambda i: (0, 0)),
            pl.BlockSpec((1, 1), lambda i: (0, 0)),
            pl.BlockSpec((d, _ER), lambda i: (0, 0)),
            pl.BlockSpec((_ER, d), lambda i: (0, 0)),
        ],
        out_specs=pl.BlockSpec((_TB, d), lambda i: (i, 0)),
        out_shape=jax.ShapeDtypeStruct((t, d), jnp.float32),
    )(flat, W_base, b_base.reshape(1, d), w_gt,
      b_thr.reshape(1, 1), a_cat, b_cat)
    return out.reshape(x.shape)
